# Initial kernel scaffold; baseline (speedup 1.0000x reference)
import jax
import jax.numpy as jnp
from jax.experimental import pallas as pl


def kernel(updates, mask):
    raise NotImplementedError("write your pallas kernel here")


# speedup vs baseline: 6.3609x; 6.3609x over previous
import functools

import jax
import jax.numpy as jnp
from jax import lax
from jax.experimental import pallas as pl
from jax.experimental.pallas import tpu as pltpu
from jax.experimental.pallas import tpu_sc as plsc

B_, H_, W_, CH = 4, 128, 128, 96
N = B_ * H_ * W_ * CH
TOTAL = N * 4
NC, NS, L = 2, 16, 16

C = 1_572_864
NPASS = 8
S_TILE = C // NS
BLK = 4_096
PER_TILE = N // NS
NBLK = PER_TILE // BLK
DUMMY_MASK = (1 << 20) - 1


def _body(upd_hbm, idx_hbm, out_hbm, acc, idxb, valb, locb, vout):
    core = lax.axis_index("c")
    sub = lax.axis_index("s")
    tile_start = sub * PER_TILE

    for p in range(NPASS):
        chunk = 2 * p + core
        base = chunk * C

        def _z(i, _):
            vout[pl.ds(i * L, L)] = jnp.zeros((L,), jnp.float32)
            return 0
        lax.fori_loop(0, BLK // L, _z, 0)
        for j in range(S_TILE // BLK):
            pltpu.sync_copy(vout, acc.at[pl.ds(sub * S_TILE + j * BLK, BLK)])
        plsc.subcore_barrier()

        def _blk(b, _):
            st = tile_start + b * BLK
            pltpu.sync_copy(idx_hbm.at[pl.ds(st, BLK)], idxb)
            pltpu.sync_copy(upd_hbm.at[pl.ds(st, BLK)], valb)

            def _vec(i, _):
                for u in range(4):
                    off = i * (4 * L) + u * L
                    vi = idxb[pl.ds(off, L)]
                    vv = valb[pl.ds(off, L)]
                    local = vi - base
                    inr = plsc.bitcast(local, jnp.uint32) < jnp.uint32(C)
                    dummy = vi & DUMMY_MASK
                    locb[pl.ds(off, L)] = jnp.where(inr, local, dummy)
                    vout[pl.ds(off, L)] = jnp.where(inr, vv, 0.0)
                return 0
            lax.fori_loop(0, BLK // (4 * L), _vec, 0)

            pltpu.sync_copy(vout, acc.at[locb], add=True)
            return 0
        lax.fori_loop(0, NBLK, _blk, 0)
        plsc.subcore_barrier()

        pltpu.sync_copy(acc.at[pl.ds(sub * S_TILE, S_TILE)],
                        out_hbm.at[pl.ds(base + sub * S_TILE, S_TILE)])
        plsc.subcore_barrier()


_scatter = pl.kernel(
    _body,
    out_type=jax.ShapeDtypeStruct((TOTAL,), jnp.float32),
    mesh=plsc.VectorSubcoreMesh(
        core_axis_name="c", subcore_axis_name="s", num_cores=NC,
        num_subcores=NS),
    scratch_types=[
        pltpu.VMEM_SHARED((C,), jnp.float32),
        pltpu.VMEM((BLK,), jnp.int32),
        pltpu.VMEM((BLK,), jnp.float32),
        pltpu.VMEM((BLK,), jnp.int32),
        pltpu.VMEM((BLK,), jnp.float32),
    ],
)


@jax.jit
def kernel(updates, mask):
    upd = updates.reshape(-1)
    idx = mask.reshape(-1).astype(jnp.int32)
    out = _scatter(upd, idx)
    return out.reshape(B_, H_ * 2, W_ * 2, CH)
